# baseline (speedup 1.0000x reference)
import functools

import jax
import jax.numpy as jnp
from jax import lax
from jax.experimental import pallas as pl
from jax.experimental.pallas import tpu as pltpu
from jax.experimental.pallas import tpu_sc as plsc

_NC, _NS = 2, 16
_NW = _NC * _NS
_CH = 16
_SC_ROWS = 1024
_TC_BLOCK = 1024


def _tc_copy_block(t_ref, o_ref):
    o_ref[...] = t_ref[...]


def _sc_part(table, T, E):
    rows_per_w = _SC_ROWS // _NW
    nch = rows_per_w // _CH
    mesh = plsc.VectorSubcoreMesh(core_axis_name="c", subcore_axis_name="s")

    @functools.partial(
        pl.kernel,
        out_type=jax.ShapeDtypeStruct((_SC_ROWS, E), table.dtype),
        mesh=mesh,
        scratch_types=[
            pltpu.VMEM((_CH, E), jnp.float32),
            pltpu.VMEM((_CH, E), jnp.float32),
            pltpu.SemaphoreType.DMA,
            pltpu.SemaphoreType.DMA,
            pltpu.SemaphoreType.DMA,
            pltpu.SemaphoreType.DMA,
        ],
    )
    def sc_copy(tbl, out, buf0, buf1, ri0, ri1, wo0, wo1):
        wid = lax.axis_index("s") * _NC + lax.axis_index("c")
        base = wid * rows_per_w
        bufs = (buf0, buf1)
        rsem = (ri0, ri1)
        wsem = (wo0, wo1)

        def rd(c):
            return pltpu.make_async_copy(
                tbl.at[pl.ds(base + c * _CH, _CH)], bufs[c % 2], rsem[c % 2]
            )

        def wr(c):
            return pltpu.make_async_copy(
                bufs[c % 2], out.at[pl.ds(base + c * _CH, _CH)], wsem[c % 2]
            )

        rd(0).start()
        for c in range(nch):
            if c + 1 < nch:
                if c - 1 >= 0:
                    wr(c - 1).wait()
                rd(c + 1).start()
            rd(c).wait()
            wr(c).start()
        if nch >= 2:
            wr(nch - 2).wait()
        wr(nch - 1).wait()

    return sc_copy(table)


def _tc_part(table, T, E):
    tc_rows = T - _SC_ROWS
    off = _SC_ROWS // _TC_BLOCK
    return pl.pallas_call(
        _tc_copy_block,
        grid=(tc_rows // _TC_BLOCK,),
        in_specs=[pl.BlockSpec((_TC_BLOCK, E), lambda i: (i + off, 0))],
        out_specs=pl.BlockSpec((_TC_BLOCK, E), lambda i: (i, 0)),
        out_shape=jax.ShapeDtypeStruct((tc_rows, E), table.dtype),
    )(table)


def kernel(x, table):
    del x
    T, E = table.shape
    sc_out = _sc_part(table, T, E)
    tc_out = _tc_part(table, T, E)
    return jnp.concatenate([sc_out, tc_out], axis=0)[None, :, :]


# speedup vs baseline: 2.8248x; 2.8248x over previous
import jax
import jax.numpy as jnp
from jax.experimental import pallas as pl

_ROWS = 2048
_COLS = 1024


def _copy_block(t_ref, o_ref):
    o_ref[...] = t_ref[...]


def kernel(x, table):
    del x
    T, E = table.shape
    out = pl.pallas_call(
        _copy_block,
        grid=(T // _ROWS, E // _COLS),
        in_specs=[pl.BlockSpec((_ROWS, _COLS), lambda i, j: (i, j))],
        out_specs=pl.BlockSpec((_ROWS, _COLS), lambda i, j: (i, j)),
        out_shape=jax.ShapeDtypeStruct((T, E), table.dtype),
    )(table)
    return out[None, :, :]
